# Initial kernel scaffold; baseline (speedup 1.0000x reference)
import jax
import jax.numpy as jnp
from jax.experimental import pallas as pl


def kernel(x, edge_index, edge_attr, emb_table, fc0_w1, fc0_w2, fc1_w1, fc1_w2, fc2_w1, fc2_w2, head_w1, head_w2):
    raise NotImplementedError("write your pallas kernel here")


# speedup vs baseline: 3.6231x; 3.6231x over previous
import functools

import jax
import jax.numpy as jnp
import numpy as np
from jax import lax
from jax.experimental import pallas as pl
from jax.experimental.pallas import tpu as pltpu
from jax.experimental.pallas import tpu_sc as plsc

N_NODES = 50000
N_EDGES = 800000
NUM_SPECIES = 100
D = 8
NUM_BASIS = 16
RADIAL_HIDDEN = 64
HEAD_HIDDEN = 64
HEAD_OUT = 4
MAX_RADIUS = 3.15

_MSG_SCALE = float(1.0 / (np.sqrt(64.0) * np.sqrt(8.0) * np.sqrt(16.0)))
_F_SCALE = float(1.0 / np.sqrt(16.0))
_HEAD1_SCALE = float(1.0 / np.sqrt(8.0))
_HEAD2_SCALE = float(1.0 / np.sqrt(64.0))

BE = 3200
BN = 5000


def _emb_body(x_ref, emb_ref, out_ref):
    xb = x_ref[...]
    iota = lax.broadcasted_iota(jnp.int32, (BN, NUM_SPECIES), 1)
    onehot = (xb == iota).astype(jnp.float32)
    out_ref[...] = jnp.dot(onehot, emb_ref[...],
                           preferred_element_type=jnp.float32)


def _embed(x, emb_table):
    x2 = x.reshape(N_NODES, 1).astype(jnp.int32)
    return pl.pallas_call(
        _emb_body,
        grid=(N_NODES // BN,),
        in_specs=[
            pl.BlockSpec((BN, 1), lambda i: (i, 0)),
            pl.BlockSpec((NUM_SPECIES, D), lambda i: (0, 0)),
        ],
        out_specs=pl.BlockSpec((BN, D), lambda i: (i, 0)),
        out_shape=jax.ShapeDtypeStruct((N_NODES, D), jnp.float32),
    )(x2, emb_table)


def _radial_basis(ea):
    step = float(MAX_RADIUS / (NUM_BASIS + 1))
    values = (lax.broadcasted_iota(jnp.int32, (1, NUM_BASIS), 1)
              + 1).astype(jnp.float32) * step
    d = jnp.sqrt(jnp.sum(ea * ea, axis=1, keepdims=True))
    diff = (d - values) * (1.0 / step)

    def sus(t):
        safe = jnp.where(t > 0.0, t, 1.0)
        return jnp.where(t > 0.0, jnp.exp(-1.0 / safe), 0.0)

    basis = (1.14136 * float(np.exp(2.0))) * sus(diff + 1.0) * sus(1.0 - diff)
    return basis * float(np.sqrt(NUM_BASIS))


def _edge_math(hb, g, w1, w2, r_mat, s_mat):
    f = jnp.maximum(
        jnp.dot(hb, w1, preferred_element_type=jnp.float32) * _F_SCALE, 0.0)
    u = jnp.dot(f, w2, preferred_element_type=jnp.float32)
    gexp = jnp.dot(g, r_mat, preferred_element_type=jnp.float32)
    return jnp.dot(gexp * u, s_mat,
                   preferred_element_type=jnp.float32) * _MSG_SCALE


def _edge0_body(ea_ref, g_ref, w1_ref, w2_ref, r_ref, s_ref,
                msg_ref, hb_ref):
    hb = _radial_basis(ea_ref[...])
    hb_ref[...] = hb
    msg_ref[...] = _edge_math(hb, g_ref[...], w1_ref[...], w2_ref[...],
                              r_ref[...], s_ref[...])


def _edge_body(hb_ref, g_ref, w1_ref, w2_ref, r_ref, s_ref, msg_ref):
    msg_ref[...] = _edge_math(hb_ref[...], g_ref[...], w1_ref[...],
                              w2_ref[...], r_ref[...], s_ref[...])


def _edge_kernel0(ea, g, w1, w2, r_mat, s_mat):
    return pl.pallas_call(
        _edge0_body,
        grid=(N_EDGES // BE,),
        in_specs=[
            pl.BlockSpec((BE, 3), lambda i: (i, 0)),
            pl.BlockSpec((BE, D), lambda i: (i, 0)),
            pl.BlockSpec((NUM_BASIS, RADIAL_HIDDEN), lambda i: (0, 0)),
            pl.BlockSpec((RADIAL_HIDDEN, D * D), lambda i: (0, 0)),
            pl.BlockSpec((D, D * D), lambda i: (0, 0)),
            pl.BlockSpec((D * D, D), lambda i: (0, 0)),
        ],
        out_specs=[
            pl.BlockSpec((BE, D), lambda i: (i, 0)),
            pl.BlockSpec((BE, NUM_BASIS), lambda i: (i, 0)),
        ],
        out_shape=[
            jax.ShapeDtypeStruct((N_EDGES, D), jnp.float32),
            jax.ShapeDtypeStruct((N_EDGES, NUM_BASIS), jnp.float32),
        ],
    )(ea, g, w1, w2, r_mat, s_mat)


def _edge_kernel(hb, g, w1, w2, r_mat, s_mat):
    return pl.pallas_call(
        _edge_body,
        grid=(N_EDGES // BE,),
        in_specs=[
            pl.BlockSpec((BE, NUM_BASIS), lambda i: (i, 0)),
            pl.BlockSpec((BE, D), lambda i: (i, 0)),
            pl.BlockSpec((NUM_BASIS, RADIAL_HIDDEN), lambda i: (0, 0)),
            pl.BlockSpec((RADIAL_HIDDEN, D * D), lambda i: (0, 0)),
            pl.BlockSpec((D, D * D), lambda i: (0, 0)),
            pl.BlockSpec((D * D, D), lambda i: (0, 0)),
        ],
        out_specs=pl.BlockSpec((BE, D), lambda i: (i, 0)),
        out_shape=jax.ShapeDtypeStruct((N_EDGES, D), jnp.float32),
    )(hb, g, w1, w2, r_mat, s_mat)


def _combine_body(a_ref, b_ref, out_ref):
    out_ref[...] = a_ref[0] + b_ref[0]


def _combine(parts):
    return pl.pallas_call(
        _combine_body,
        grid=(N_NODES // BN,),
        in_specs=[
            pl.BlockSpec((1, BN, D), lambda i: (0, i, 0)),
            pl.BlockSpec((1, BN, D), lambda i: (1, i, 0)),
        ],
        out_specs=pl.BlockSpec((BN, D), lambda i: (i, 0)),
        out_shape=jax.ShapeDtypeStruct((N_NODES, D), jnp.float32),
    )(parts, parts)


def _head_body(a_ref, b_ref, w1_ref, w2_ref, out_ref):
    h = a_ref[0] + b_ref[0]
    z = jnp.dot(h, w1_ref[...], preferred_element_type=jnp.float32)
    z = z * _HEAD1_SCALE
    hid = z / (1.0 + jnp.exp(-z))
    out_ref[...] = jnp.dot(hid, w2_ref[...],
                           preferred_element_type=jnp.float32) * _HEAD2_SCALE


def _head(parts, head_w1, head_w2):
    return pl.pallas_call(
        _head_body,
        grid=(N_NODES // BN,),
        in_specs=[
            pl.BlockSpec((1, BN, D), lambda i: (0, i, 0)),
            pl.BlockSpec((1, BN, D), lambda i: (1, i, 0)),
            pl.BlockSpec((D, HEAD_HIDDEN), lambda i: (0, 0)),
            pl.BlockSpec((HEAD_HIDDEN, HEAD_OUT), lambda i: (0, 0)),
        ],
        out_specs=pl.BlockSpec((BN, HEAD_OUT), lambda i: (i, 0)),
        out_shape=jax.ShapeDtypeStruct((N_NODES, HEAD_OUT), jnp.float32),
    )(parts, parts, head_w1, head_w2)


_NC = 2
_NS = 16
_NW = _NC * _NS

_G_CH = 5000
_G_PER_W = N_EDGES // _NW
_G_NCH = _G_PER_W // _G_CH

_S_CH = 5000
_S_PER_T = (N_EDGES // _NC) // _NS
_S_NCH = _S_PER_T // _S_CH
_ROWS_PER_T = N_NODES // _NS


def _gather_body(table_hbm, idx_hbm, out_hbm, idx_v, rows_v, sem):
    wid = lax.axis_index("s") * _NC + lax.axis_index("c")
    base0 = wid * _G_PER_W
    for k in range(_G_NCH):
        base = base0 + k * _G_CH
        pltpu.sync_copy(idx_hbm.at[pl.ds(base, _G_CH)], idx_v)
        pltpu.async_copy(table_hbm.at[idx_v], rows_v, sem).wait()
        pltpu.sync_copy(rows_v, out_hbm.at[pl.ds(base, _G_CH)])


@functools.cache
def _gather_call():
    return pl.kernel(
        _gather_body,
        out_type=jax.ShapeDtypeStruct((N_EDGES, D), jnp.float32),
        mesh=plsc.VectorSubcoreMesh(core_axis_name="c", subcore_axis_name="s",
                                    num_cores=_NC, num_subcores=_NS),
        scratch_types=[
            pltpu.VMEM((_G_CH,), jnp.int32),
            pltpu.VMEM((_G_CH, D), jnp.float32),
            pltpu.SemaphoreType.DMA,
        ],
        compiler_params=pltpu.CompilerParams(use_tc_tiling_on_sc=False),
    )


def _scatter_body(msg_hbm, dst_hbm, zeros_hbm, out_hbm, idx_v, msg_v, acc):
    cid = lax.axis_index("c")
    sid = lax.axis_index("s")
    row0 = sid * _ROWS_PER_T
    pltpu.sync_copy(zeros_hbm, acc.at[pl.ds(row0, _ROWS_PER_T)])
    plsc.subcore_barrier()
    base0 = cid * (N_EDGES // _NC) + sid * _S_PER_T
    for k in range(_S_NCH):
        base = base0 + k * _S_CH
        pltpu.sync_copy(dst_hbm.at[pl.ds(base, _S_CH)], idx_v)
        pltpu.sync_copy(msg_hbm.at[pl.ds(base, _S_CH)], msg_v)
        pltpu.sync_copy(msg_v, acc.at[idx_v], add=True)
    plsc.subcore_barrier()
    pltpu.sync_copy(acc.at[pl.ds(row0, _ROWS_PER_T)],
                    out_hbm.at[cid, pl.ds(row0, _ROWS_PER_T)])


@functools.cache
def _scatter_call():
    return pl.kernel(
        _scatter_body,
        out_type=jax.ShapeDtypeStruct((_NC, N_NODES, D), jnp.float32),
        mesh=plsc.VectorSubcoreMesh(core_axis_name="c", subcore_axis_name="s",
                                    num_cores=_NC, num_subcores=_NS),
        scratch_types=[
            pltpu.VMEM((_S_CH,), jnp.int32),
            pltpu.VMEM((_S_CH, D), jnp.float32),
            pltpu.VMEM_SHARED((N_NODES, D), jnp.float32),
        ],
        compiler_params=pltpu.CompilerParams(use_tc_tiling_on_sc=False),
    )


def kernel(x, edge_index, edge_attr, emb_table,
           fc0_w1, fc0_w2, fc1_w1, fc1_w2, fc2_w1, fc2_w2,
           head_w1, head_w2):
    src = edge_index[0].astype(jnp.int32)
    dst = edge_index[1].astype(jnp.int32)

    r_mat = jnp.asarray(np.repeat(np.eye(D, dtype=np.float32), D, axis=1))
    s_mat = jnp.asarray(np.tile(np.eye(D, dtype=np.float32), (D, 1)))
    zeros_rows = jnp.zeros((_ROWS_PER_T, D), jnp.float32)

    h = _embed(x, emb_table)

    hb = None
    for layer, (w1, w2) in enumerate(
            ((fc0_w1, fc0_w2), (fc1_w1, fc1_w2), (fc2_w1, fc2_w2))):
        g = _gather_call()(h, src)
        if layer == 0:
            msg, hb = _edge_kernel0(edge_attr, g, w1, w2, r_mat, s_mat)
        else:
            msg = _edge_kernel(hb, g, w1, w2, r_mat, s_mat)
        parts = _scatter_call()(msg, dst, zeros_rows)
        if layer < 2:
            h = _combine(parts)

    return _head(parts, head_w1, head_w2)


# speedup vs baseline: 5.4189x; 1.4957x over previous
import functools

import jax
import jax.numpy as jnp
import numpy as np
from jax import lax
from jax.experimental import pallas as pl
from jax.experimental.pallas import tpu as pltpu
from jax.experimental.pallas import tpu_sc as plsc

N_NODES = 50000
N_EDGES = 800000
NUM_SPECIES = 100
D = 8
NUM_BASIS = 16
RADIAL_HIDDEN = 64
HEAD_HIDDEN = 64
HEAD_OUT = 4
MAX_RADIUS = 3.15

_MSG_SCALE = float(1.0 / (np.sqrt(64.0) * np.sqrt(8.0) * np.sqrt(16.0)))
_F_SCALE = float(1.0 / np.sqrt(16.0))
_HEAD1_SCALE = float(1.0 / np.sqrt(8.0))
_HEAD2_SCALE = float(1.0 / np.sqrt(64.0))

BEL = 6400

_F32 = jnp.float32
_SDS = jax.ShapeDtypeStruct


def _radial_basis_t(ea_blk):
    step = float(MAX_RADIUS / (NUM_BASIS + 1))
    eat = jnp.transpose(ea_blk, (1, 0))
    d = jnp.sqrt(jnp.sum(eat * eat, axis=0, keepdims=True))
    vals = (lax.broadcasted_iota(jnp.int32, (NUM_BASIS, BEL), 0)
            + 1).astype(_F32) * step
    diff = (d - vals) * (1.0 / step)

    def sus(t):
        safe = jnp.where(t > 0.0, t, 1.0)
        return jnp.where(t > 0.0, jnp.exp(-1.0 / safe), 0.0)

    basis = (1.14136 * float(np.exp(2.0))) * sus(diff + 1.0) * sus(1.0 - diff)
    return basis * float(np.sqrt(NUM_BASIS))


def _edge_math_t(hbT, g_vals, w1t, w2t):
    fT = jnp.maximum(
        jnp.dot(w1t, hbT, preferred_element_type=_F32) * _F_SCALE, 0.0)
    uT = jnp.dot(w2t, fT, preferred_element_type=_F32)
    acc = jnp.zeros((D, BEL), _F32)
    for i in range(D):
        gb = jnp.broadcast_to(g_vals[i], (D, BEL))
        acc = acc + gb * uT[D * i:D * i + D, :]
    return acc * _MSG_SCALE


def _edge0_body(*refs):
    ea_ref = refs[0]
    g_refs = refs[1:9]
    w1t_ref, w2t_ref = refs[9], refs[10]
    msg_refs = refs[11:19]
    hbt_ref = refs[19]
    hbT = _radial_basis_t(ea_ref[...])
    hbt_ref[...] = hbT
    msgT = _edge_math_t(hbT, [r[...] for r in g_refs],
                        w1t_ref[...], w2t_ref[...])
    for o in range(D):
        msg_refs[o][...] = msgT[o:o + 1, :]


def _edge_body(*refs):
    hbt_ref = refs[0]
    g_refs = refs[1:9]
    w1t_ref, w2t_ref = refs[9], refs[10]
    msg_refs = refs[11:19]
    msgT = _edge_math_t(hbt_ref[...], [r[...] for r in g_refs],
                        w1t_ref[...], w2t_ref[...])
    for o in range(D):
        msg_refs[o][...] = msgT[o:o + 1, :]


_G_SPEC = pl.BlockSpec((1, BEL), lambda i: (0, i))
_W1T_SPEC = pl.BlockSpec((RADIAL_HIDDEN, NUM_BASIS), lambda i: (0, 0))
_W2T_SPEC = pl.BlockSpec((RADIAL_HIDDEN, RADIAL_HIDDEN), lambda i: (0, 0))
_MSG_SHAPE = [_SDS((1, N_EDGES), _F32) for _ in range(D)]
_MSG_SPECS = [_G_SPEC] * D


def _edge_kernel0(ea, g_list, w1t, w2t):
    return pl.pallas_call(
        _edge0_body,
        grid=(N_EDGES // BEL,),
        in_specs=[pl.BlockSpec((BEL, 3), lambda i: (i, 0))]
        + [_G_SPEC] * D + [_W1T_SPEC, _W2T_SPEC],
        out_specs=_MSG_SPECS + [pl.BlockSpec((NUM_BASIS, BEL),
                                             lambda i: (0, i))],
        out_shape=_MSG_SHAPE + [_SDS((NUM_BASIS, N_EDGES), _F32)],
    )(ea, *g_list, w1t, w2t)


def _edge_kernel(hbt, g_list, w1t, w2t):
    return pl.pallas_call(
        _edge_body,
        grid=(N_EDGES // BEL,),
        in_specs=[pl.BlockSpec((NUM_BASIS, BEL), lambda i: (0, i))]
        + [_G_SPEC] * D + [_W1T_SPEC, _W2T_SPEC],
        out_specs=_MSG_SPECS,
        out_shape=_MSG_SHAPE,
    )(hbt, *g_list, w1t, w2t)


def _combine_body(a_ref, b_ref, out_ref):
    out_ref[...] = a_ref[...] + b_ref[...]


def _combine(p0_flat, p1_flat):
    return pl.pallas_call(
        _combine_body,
        out_shape=_SDS((N_NODES * D,), _F32),
    )(p0_flat, p1_flat)


def _head_body(*refs):
    p0 = refs[0:D]
    p1 = refs[D:2 * D]
    w1t_ref, w2t_ref = refs[2 * D], refs[2 * D + 1]
    out_ref = refs[2 * D + 2]
    hT = jnp.concatenate([p0[i][...] + p1[i][...] for i in range(D)],
                         axis=0)
    z = jnp.dot(w1t_ref[...], hT, preferred_element_type=_F32) * _HEAD1_SCALE
    hid = z / (1.0 + jnp.exp(-z))
    out_ref[...] = jnp.dot(w2t_ref[...], hid,
                           preferred_element_type=_F32) * _HEAD2_SCALE


def _head(pc_list, head_w1t, head_w2t):
    return pl.pallas_call(
        _head_body,
        out_shape=_SDS((HEAD_OUT, N_NODES), _F32),
    )(*pc_list, head_w1t, head_w2t)


_NC = 2
_NS = 16
_NW = _NC * _NS

_CH = 5000
_CHP = 5008
_NGR = 313
_PER_W = N_EDGES // _NW
_NCH_G = _PER_W // _CH
_PER_T = (N_EDGES // _NC) // _NS
_NCH_S = _PER_T // _CH
_ZROWS = N_NODES // _NS
_TROWS = 3128
_TROWS_LAST = N_NODES - 15 * _TROWS

_SC_PARAMS = pltpu.CompilerParams(use_tc_tiling_on_sc=False,
                                  needs_layout_passes=False)
def _sc_mesh():
    return plsc.VectorSubcoreMesh(
        core_axis_name="c", subcore_axis_name="s",
        num_cores=_NC, num_subcores=_NS)


def _deinterleave(rows_v, comp_v, ngroups):
    def body(k, _):
        t = k * 16
        ridx = t + lax.iota(jnp.int32, 16)
        for i in range(D):
            cidx = jnp.full((16,), i, jnp.int32)
            comp_v[i, pl.ds(t, 16)] = plsc.load_gather(rows_v, [ridx, cidx])
        return 0
    lax.fori_loop(0, ngroups, body, 0)


def _gather_body(table_hbm, idx_hbm, *refs):
    out_refs = refs[0:D]
    idx_v, rows_v, comp_v, sem = refs[D:D + 4]
    wid = lax.axis_index("s") * _NC + lax.axis_index("c")
    for k in range(_NCH_G):
        base = wid * _PER_W + k * _CH
        pltpu.sync_copy(idx_hbm.at[pl.ds(base, _CH)],
                        idx_v.at[pl.ds(0, _CH)])
        pltpu.async_copy(table_hbm.at[idx_v.at[pl.ds(0, _CH)]],
                         rows_v.at[pl.ds(0, _CH)], sem).wait()
        _deinterleave(rows_v, comp_v, _NGR)
        for i in range(D):
            pltpu.sync_copy(comp_v.at[i, pl.ds(0, _CH)],
                            out_refs[i].at[0, pl.ds(base, _CH)])


@functools.cache
def _gather_call():
    return pl.kernel(
        _gather_body,
        out_type=[_SDS((1, N_EDGES), _F32) for _ in range(D)],
        scratch_types=[
            pltpu.VMEM((_CHP,), jnp.int32),
            pltpu.VMEM((_CHP, D), _F32),
            pltpu.VMEM((D, _CHP), _F32),
            pltpu.SemaphoreType.DMA,
        ],
        compiler_params=_SC_PARAMS,
        mesh=_sc_mesh(),
    )


def _emb_gather_body(x_hbm, emb_hbm, idx_hbm, *refs):
    out_refs = refs[0:D]
    x_v, emb_v, idx_v, comp_v = refs[D:D + 4]
    wid = lax.axis_index("s") * _NC + lax.axis_index("c")
    pltpu.sync_copy(x_hbm, x_v)
    pltpu.sync_copy(emb_hbm, emb_v)
    for k in range(_NCH_G):
        base = wid * _PER_W + k * _CH
        pltpu.sync_copy(idx_hbm.at[pl.ds(base, _CH)],
                        idx_v.at[pl.ds(0, _CH)])

        def body(kk, _):
            t = kk * 16
            srcv = idx_v[pl.ds(t, 16)]
            spec = plsc.load_gather(x_v, [srcv])
            for i in range(D):
                cidx = jnp.full((16,), i, jnp.int32)
                comp_v[i, pl.ds(t, 16)] = plsc.load_gather(
                    emb_v, [spec, cidx])
            return 0
        lax.fori_loop(0, _NGR - 1, body, 0)
        t = (_NGR - 1) * 16
        lane = t + lax.iota(jnp.int32, 16)
        lane = jnp.minimum(lane, _CH - 1)
        srcv = plsc.load_gather(idx_v, [lane])
        spec = plsc.load_gather(x_v, [srcv])
        for i in range(D):
            cidx = jnp.full((16,), i, jnp.int32)
            comp_v[i, pl.ds(t, 16)] = plsc.load_gather(emb_v, [spec, cidx])
        for i in range(D):
            pltpu.sync_copy(comp_v.at[i, pl.ds(0, _CH)],
                            out_refs[i].at[0, pl.ds(base, _CH)])


@functools.cache
def _emb_gather_call():
    return pl.kernel(
        _emb_gather_body,
        out_type=[_SDS((1, N_EDGES), _F32) for _ in range(D)],
        scratch_types=[
            pltpu.VMEM((N_NODES,), jnp.int32),
            pltpu.VMEM((NUM_SPECIES, D), _F32),
            pltpu.VMEM((_CHP,), jnp.int32),
            pltpu.VMEM((D, _CHP), _F32),
        ],
        compiler_params=_SC_PARAMS,
        mesh=_sc_mesh(),
    )


def _interleave(comp_v, rows_v, ngroups):
    def body(k, _):
        t = k * 16
        ridx = t + lax.iota(jnp.int32, 16)
        for i in range(D):
            cidx = jnp.full((16,), i, jnp.int32)
            plsc.store_scatter(rows_v, [ridx, cidx],
                               comp_v[i, pl.ds(t, 16)])
        return 0
    lax.fori_loop(0, ngroups, body, 0)


def _scatter_common(msg_refs, dst_hbm, zeros_hbm, idx_v, comp_v, rows_v, acc):
    cid = lax.axis_index("c")
    sid = lax.axis_index("s")
    pltpu.sync_copy(zeros_hbm, acc.at[pl.ds(sid * _ZROWS, _ZROWS)])
    plsc.subcore_barrier()
    for k in range(_NCH_S):
        base = cid * (N_EDGES // _NC) + sid * _PER_T + k * _CH
        pltpu.sync_copy(dst_hbm.at[pl.ds(base, _CH)], idx_v)
        for i in range(D):
            pltpu.sync_copy(msg_refs[i].at[0, pl.ds(base, _CH)],
                            comp_v.at[i, pl.ds(0, _CH)])
        _interleave(comp_v, rows_v, _NGR)
        pltpu.sync_copy(rows_v.at[pl.ds(0, _CH)], acc.at[idx_v], add=True)
    plsc.subcore_barrier()
    return cid, sid


def _scatter_body(*refs):
    msg_refs = refs[0:D]
    dst_hbm, zeros_hbm, out_hbm = refs[D], refs[D + 1], refs[D + 2]
    idx_v, comp_v, rows_v, acc = refs[D + 3:D + 7]
    cid, sid = _scatter_common(msg_refs, dst_hbm, zeros_hbm,
                               idx_v, comp_v, rows_v, acc)
    row0 = sid * _ZROWS
    pltpu.sync_copy(acc.at[pl.ds(row0, _ZROWS)],
                    out_hbm.at[cid, pl.ds(row0, _ZROWS)])


def _scatter_t_body(*refs):
    msg_refs = refs[0:D]
    dst_hbm, zeros_hbm = refs[D], refs[D + 1]
    out_refs = refs[D + 2:D + 2 + 2 * D]
    idx_v, comp_v, rows_v, acc = refs[D + 2 + 2 * D:]
    cid, sid = _scatter_common(msg_refs, dst_hbm, zeros_hbm,
                               idx_v, comp_v, rows_v, acc)

    def writeout(row0, cnt, ngroups):
        pltpu.sync_copy(acc.at[pl.ds(row0, cnt)], rows_v.at[pl.ds(0, cnt)])
        _deinterleave(rows_v, comp_v, ngroups)
        for o in range(D):
            for c in range(_NC):
                @pl.when(cid == c)
                def _():
                    pltpu.sync_copy(comp_v.at[o, pl.ds(0, cnt)],
                                    out_refs[c * D + o].at[0,
                                                           pl.ds(row0, cnt)])

    @pl.when(sid < _NS - 1)
    def _():
        writeout(sid * _TROWS, _TROWS, (_TROWS + 15) // 16)

    @pl.when(sid == _NS - 1)
    def _():
        writeout((_NS - 1) * _TROWS, _TROWS_LAST, (_TROWS_LAST + 15) // 16)


_SC_SCRATCH = [
    pltpu.VMEM((_CH,), jnp.int32),
    pltpu.VMEM((D, _CHP), _F32),
    pltpu.VMEM((_CHP, D), _F32),
    pltpu.VMEM_SHARED((N_NODES, D), _F32),
]


@functools.cache
def _scatter_call():
    return pl.kernel(
        _scatter_body,
        out_type=_SDS((_NC, N_NODES, D), _F32),
        scratch_types=_SC_SCRATCH,
        compiler_params=_SC_PARAMS,
        mesh=_sc_mesh(),
    )


@functools.cache
def _scatter_t_call():
    return pl.kernel(
        _scatter_t_body,
        out_type=[_SDS((1, N_NODES), _F32) for _ in range(2 * D)],
        scratch_types=_SC_SCRATCH,
        compiler_params=_SC_PARAMS,
        mesh=_sc_mesh(),
    )


def kernel(x, edge_index, edge_attr, emb_table,
           fc0_w1, fc0_w2, fc1_w1, fc1_w2, fc2_w1, fc2_w2,
           head_w1, head_w2):
    src = edge_index[0].astype(jnp.int32)
    dst = edge_index[1].astype(jnp.int32)
    zeros_rows = jnp.zeros((_ZROWS, D), _F32)
    xi = x.astype(jnp.int32)

    w1t = (jnp.transpose(fc0_w1), jnp.transpose(fc1_w1),
           jnp.transpose(fc2_w1))
    w2t = (jnp.transpose(fc0_w2), jnp.transpose(fc1_w2),
           jnp.transpose(fc2_w2))

    hbt = None
    for layer in range(3):
        if layer == 0:
            g_list = _emb_gather_call()(xi, emb_table, src)
            outs = _edge_kernel0(edge_attr, g_list, w1t[0], w2t[0])
            msg_list, hbt = outs[:D], outs[D]
        else:
            g_list = _gather_call()(h, src)
            msg_list = _edge_kernel(hbt, g_list, w1t[layer], w2t[layer])
        if layer < 2:
            parts = _scatter_call()(*msg_list, dst, zeros_rows)
            parts_flat = parts.reshape(_NC, N_NODES * D)
            h = _combine(parts_flat[0], parts_flat[1]).reshape(N_NODES, D)
        else:
            pc_list = _scatter_t_call()(*msg_list, dst, zeros_rows)

    out_t = _head(pc_list, jnp.transpose(head_w1), jnp.transpose(head_w2))
    return jnp.transpose(out_t)


# speedup vs baseline: 5.7538x; 1.0618x over previous
import functools

import jax
import jax.numpy as jnp
import numpy as np
from jax import lax
from jax.experimental import pallas as pl
from jax.experimental.pallas import tpu as pltpu
from jax.experimental.pallas import tpu_sc as plsc

N_NODES = 50000
N_EDGES = 800000
NUM_SPECIES = 100
D = 8
NUM_BASIS = 16
RADIAL_HIDDEN = 64
HEAD_HIDDEN = 64
HEAD_OUT = 4
MAX_RADIUS = 3.15

_MSG_SCALE = float(1.0 / (np.sqrt(64.0) * np.sqrt(8.0) * np.sqrt(16.0)))
_F_SCALE = float(1.0 / np.sqrt(16.0))
_HEAD1_SCALE = float(1.0 / np.sqrt(8.0))
_HEAD2_SCALE = float(1.0 / np.sqrt(64.0))

BEL = 6400

_F32 = jnp.float32
_SDS = jax.ShapeDtypeStruct


def _radial_basis_t(eat):
    step = float(MAX_RADIUS / (NUM_BASIS + 1))
    d = jnp.sqrt(jnp.sum(eat * eat, axis=0, keepdims=True))
    vals = (lax.broadcasted_iota(jnp.int32, (NUM_BASIS, BEL), 0)
            + 1).astype(_F32) * step
    diff = (d - vals) * (1.0 / step)

    def sus(t):
        safe = jnp.where(t > 0.0, t, 1.0)
        return jnp.where(t > 0.0, jnp.exp(-1.0 / safe), 0.0)

    basis = (1.14136 * float(np.exp(2.0))) * sus(diff + 1.0) * sus(1.0 - diff)
    return basis * float(np.sqrt(NUM_BASIS))


def _edge_math_t(hbT, g_vals, w1t, w2t):
    fT = jnp.maximum(
        jnp.dot(w1t, hbT, preferred_element_type=_F32) * _F_SCALE, 0.0)
    uT = jnp.dot(w2t, fT, preferred_element_type=_F32)
    acc = jnp.zeros((D, BEL), _F32)
    for i in range(D):
        gb = jnp.broadcast_to(g_vals[i], (D, BEL))
        acc = acc + gb * uT[D * i:D * i + D, :]
    return acc * _MSG_SCALE


def _edge0_body(*refs):
    ea_ref = refs[0]
    g_refs = refs[1:9]
    w1t_ref, w2t_ref = refs[9], refs[10]
    msg_refs = refs[11:19]
    hbt_ref = refs[19]
    hbT = _radial_basis_t(ea_ref[...])
    hbt_ref[...] = hbT
    msgT = _edge_math_t(hbT, [r[...] for r in g_refs],
                        w1t_ref[...], w2t_ref[...])
    for o in range(D):
        msg_refs[o][...] = msgT[o:o + 1, :]


def _edge_body(*refs):
    hbt_ref = refs[0]
    g_refs = refs[1:9]
    w1t_ref, w2t_ref = refs[9], refs[10]
    msg_refs = refs[11:19]
    msgT = _edge_math_t(hbt_ref[...], [r[...] for r in g_refs],
                        w1t_ref[...], w2t_ref[...])
    for o in range(D):
        msg_refs[o][...] = msgT[o:o + 1, :]


_G_SPEC = pl.BlockSpec((1, BEL), lambda i: (0, i))
_W1T_SPEC = pl.BlockSpec((RADIAL_HIDDEN, NUM_BASIS), lambda i: (0, 0))
_W2T_SPEC = pl.BlockSpec((RADIAL_HIDDEN, RADIAL_HIDDEN), lambda i: (0, 0))
_MSG_SHAPE = [_SDS((1, N_EDGES), _F32) for _ in range(D)]
_MSG_SPECS = [_G_SPEC] * D


def _edge_kernel0(ea, g_list, w1t, w2t):
    return pl.pallas_call(
        _edge0_body,
        grid=(N_EDGES // BEL,),
        in_specs=[pl.BlockSpec((3, BEL), lambda i: (0, i))]
        + [_G_SPEC] * D + [_W1T_SPEC, _W2T_SPEC],
        out_specs=_MSG_SPECS + [pl.BlockSpec((NUM_BASIS, BEL),
                                             lambda i: (0, i))],
        out_shape=_MSG_SHAPE + [_SDS((NUM_BASIS, N_EDGES), _F32)],
    )(ea, *g_list, w1t, w2t)


def _edge_kernel(hbt, g_list, w1t, w2t):
    return pl.pallas_call(
        _edge_body,
        grid=(N_EDGES // BEL,),
        in_specs=[pl.BlockSpec((NUM_BASIS, BEL), lambda i: (0, i))]
        + [_G_SPEC] * D + [_W1T_SPEC, _W2T_SPEC],
        out_specs=_MSG_SPECS,
        out_shape=_MSG_SHAPE,
    )(hbt, *g_list, w1t, w2t)


def _combine_body(a_ref, b_ref, out_ref):
    out_ref[...] = a_ref[...] + b_ref[...]


def _combine(p0_flat, p1_flat):
    return pl.pallas_call(
        _combine_body,
        out_shape=_SDS((N_NODES * D,), _F32),
    )(p0_flat, p1_flat)


def _head_body(*refs):
    p0 = refs[0:D]
    p1 = refs[D:2 * D]
    w1t_ref, w2t_ref = refs[2 * D], refs[2 * D + 1]
    out_ref = refs[2 * D + 2]
    hT = jnp.concatenate([p0[i][...] + p1[i][...] for i in range(D)],
                         axis=0)
    z = jnp.dot(w1t_ref[...], hT, preferred_element_type=_F32) * _HEAD1_SCALE
    hid = z / (1.0 + jnp.exp(-z))
    out_ref[...] = jnp.dot(w2t_ref[...], hid,
                           preferred_element_type=_F32) * _HEAD2_SCALE


def _head(pc_list, head_w1t, head_w2t):
    return pl.pallas_call(
        _head_body,
        out_shape=_SDS((HEAD_OUT, N_NODES), _F32),
    )(*pc_list, head_w1t, head_w2t)


_NC = 2
_NS = 16
_NW = _NC * _NS

_CH = 5000
_CHP = 5008
_NGR = 313
_PER_W = N_EDGES // _NW
_NCH_G = _PER_W // _CH
_PER_T = (N_EDGES // _NC) // _NS
_NCH_S = _PER_T // _CH
_ZROWS = N_NODES // _NS
_TROWS = 3128
_TROWS_LAST = N_NODES - 15 * _TROWS

_SC_PARAMS = pltpu.CompilerParams(use_tc_tiling_on_sc=False,
                                  needs_layout_passes=False)
def _sc_mesh():
    return plsc.VectorSubcoreMesh(
        core_axis_name="c", subcore_axis_name="s",
        num_cores=_NC, num_subcores=_NS)


def _deinterleave(rows_v, comp_v, ngroups):
    def body(k, _):
        t = k * 16
        ridx = t + lax.iota(jnp.int32, 16)
        for i in range(D):
            cidx = jnp.full((16,), i, jnp.int32)
            comp_v[i, pl.ds(t, 16)] = plsc.load_gather(rows_v, [ridx, cidx])
        return 0
    lax.fori_loop(0, ngroups, body, 0)


def _gather_body(table_hbm, idx_hbm, *refs):
    out_refs = refs[0:D]
    idx_v, rows_v, comp_v, sem = refs[D:D + 4]
    wid = lax.axis_index("s") * _NC + lax.axis_index("c")
    for k in range(_NCH_G):
        base = wid * _PER_W + k * _CH
        pltpu.sync_copy(idx_hbm.at[pl.ds(base, _CH)],
                        idx_v.at[pl.ds(0, _CH)])
        pltpu.async_copy(table_hbm.at[idx_v.at[pl.ds(0, _CH)]],
                         rows_v.at[pl.ds(0, _CH)], sem).wait()
        _deinterleave(rows_v, comp_v, _NGR)
        for i in range(D):
            pltpu.sync_copy(comp_v.at[i, pl.ds(0, _CH)],
                            out_refs[i].at[0, pl.ds(base, _CH)])


@functools.cache
def _gather_call():
    return pl.kernel(
        _gather_body,
        out_type=[_SDS((1, N_EDGES), _F32) for _ in range(D)],
        scratch_types=[
            pltpu.VMEM((_CHP,), jnp.int32),
            pltpu.VMEM((_CHP, D), _F32),
            pltpu.VMEM((D, _CHP), _F32),
            pltpu.SemaphoreType.DMA,
        ],
        compiler_params=_SC_PARAMS,
        mesh=_sc_mesh(),
    )


def _emb_gather_body(x_hbm, emb_hbm, idx_hbm, *refs):
    out_refs = refs[0:D]
    x_v, emb_v, idx_v, comp_v = refs[D:D + 4]
    wid = lax.axis_index("s") * _NC + lax.axis_index("c")
    pltpu.sync_copy(x_hbm, x_v)
    pltpu.sync_copy(emb_hbm, emb_v)
    for k in range(_NCH_G):
        base = wid * _PER_W + k * _CH
        pltpu.sync_copy(idx_hbm.at[pl.ds(base, _CH)],
                        idx_v.at[pl.ds(0, _CH)])

        def body(kk, _):
            t = kk * 16
            srcv = idx_v[pl.ds(t, 16)]
            spec = plsc.load_gather(x_v, [srcv])
            for i in range(D):
                cidx = jnp.full((16,), i, jnp.int32)
                comp_v[i, pl.ds(t, 16)] = plsc.load_gather(
                    emb_v, [spec, cidx])
            return 0
        lax.fori_loop(0, _NGR - 1, body, 0)
        t = (_NGR - 1) * 16
        lane = t + lax.iota(jnp.int32, 16)
        lane = jnp.minimum(lane, _CH - 1)
        srcv = plsc.load_gather(idx_v, [lane])
        spec = plsc.load_gather(x_v, [srcv])
        for i in range(D):
            cidx = jnp.full((16,), i, jnp.int32)
            comp_v[i, pl.ds(t, 16)] = plsc.load_gather(emb_v, [spec, cidx])
        for i in range(D):
            pltpu.sync_copy(comp_v.at[i, pl.ds(0, _CH)],
                            out_refs[i].at[0, pl.ds(base, _CH)])


@functools.cache
def _emb_gather_call():
    return pl.kernel(
        _emb_gather_body,
        out_type=[_SDS((1, N_EDGES), _F32) for _ in range(D)],
        scratch_types=[
            pltpu.VMEM((N_NODES,), jnp.int32),
            pltpu.VMEM((NUM_SPECIES, D), _F32),
            pltpu.VMEM((_CHP,), jnp.int32),
            pltpu.VMEM((D, _CHP), _F32),
        ],
        compiler_params=_SC_PARAMS,
        mesh=_sc_mesh(),
    )


def _interleave(comp_v, rows_v, ngroups):
    def body(k, _):
        t = k * 16
        ridx = t + lax.iota(jnp.int32, 16)
        for i in range(D):
            cidx = jnp.full((16,), i, jnp.int32)
            plsc.store_scatter(rows_v, [ridx, cidx],
                               comp_v[i, pl.ds(t, 16)])
        return 0
    lax.fori_loop(0, ngroups, body, 0)


def _scatter_common(msg_refs, dst_hbm, zeros_hbm, idx_v, comp_v, rows_v, acc):
    cid = lax.axis_index("c")
    sid = lax.axis_index("s")
    pltpu.sync_copy(zeros_hbm, acc.at[pl.ds(sid * _ZROWS, _ZROWS)])
    plsc.subcore_barrier()
    for k in range(_NCH_S):
        base = cid * (N_EDGES // _NC) + sid * _PER_T + k * _CH
        pltpu.sync_copy(dst_hbm.at[pl.ds(base, _CH)], idx_v)
        for i in range(D):
            pltpu.sync_copy(msg_refs[i].at[0, pl.ds(base, _CH)],
                            comp_v.at[i, pl.ds(0, _CH)])
        _interleave(comp_v, rows_v, _NGR)
        pltpu.sync_copy(rows_v.at[pl.ds(0, _CH)], acc.at[idx_v], add=True)
    plsc.subcore_barrier()
    return cid, sid


def _scatter_body(*refs):
    msg_refs = refs[0:D]
    dst_hbm, zeros_hbm, out_hbm = refs[D], refs[D + 1], refs[D + 2]
    idx_v, comp_v, rows_v, acc = refs[D + 3:D + 7]
    cid, sid = _scatter_common(msg_refs, dst_hbm, zeros_hbm,
                               idx_v, comp_v, rows_v, acc)
    row0 = sid * _ZROWS
    pltpu.sync_copy(acc.at[pl.ds(row0, _ZROWS)],
                    out_hbm.at[cid, pl.ds(row0, _ZROWS)])


def _scatter_t_body(*refs):
    msg_refs = refs[0:D]
    dst_hbm, zeros_hbm = refs[D], refs[D + 1]
    out_refs = refs[D + 2:D + 2 + 2 * D]
    idx_v, comp_v, rows_v, acc = refs[D + 2 + 2 * D:]
    cid, sid = _scatter_common(msg_refs, dst_hbm, zeros_hbm,
                               idx_v, comp_v, rows_v, acc)

    def writeout(row0, cnt, ngroups):
        pltpu.sync_copy(acc.at[pl.ds(row0, cnt)], rows_v.at[pl.ds(0, cnt)])
        _deinterleave(rows_v, comp_v, ngroups)
        for o in range(D):
            for c in range(_NC):
                @pl.when(cid == c)
                def _():
                    pltpu.sync_copy(comp_v.at[o, pl.ds(0, cnt)],
                                    out_refs[c * D + o].at[0,
                                                           pl.ds(row0, cnt)])

    @pl.when(sid < _NS - 1)
    def _():
        writeout(sid * _TROWS, _TROWS, (_TROWS + 15) // 16)

    @pl.when(sid == _NS - 1)
    def _():
        writeout((_NS - 1) * _TROWS, _TROWS_LAST, (_TROWS_LAST + 15) // 16)


_SC_SCRATCH = [
    pltpu.VMEM((_CH,), jnp.int32),
    pltpu.VMEM((D, _CHP), _F32),
    pltpu.VMEM((_CHP, D), _F32),
    pltpu.VMEM_SHARED((N_NODES, D), _F32),
]


@functools.cache
def _scatter_call():
    return pl.kernel(
        _scatter_body,
        out_type=_SDS((_NC, N_NODES, D), _F32),
        scratch_types=_SC_SCRATCH,
        compiler_params=_SC_PARAMS,
        mesh=_sc_mesh(),
    )


@functools.cache
def _scatter_t_call():
    return pl.kernel(
        _scatter_t_body,
        out_type=[_SDS((1, N_NODES), _F32) for _ in range(2 * D)],
        scratch_types=_SC_SCRATCH,
        compiler_params=_SC_PARAMS,
        mesh=_sc_mesh(),
    )


def kernel(x, edge_index, edge_attr, emb_table,
           fc0_w1, fc0_w2, fc1_w1, fc1_w2, fc2_w1, fc2_w2,
           head_w1, head_w2):
    src = edge_index[0].astype(jnp.int32)
    dst = edge_index[1].astype(jnp.int32)
    zeros_rows = jnp.zeros((_ZROWS, D), _F32)
    xi = x.astype(jnp.int32)
    eat = jnp.transpose(edge_attr)

    w1t = (jnp.transpose(fc0_w1), jnp.transpose(fc1_w1),
           jnp.transpose(fc2_w1))
    w2t = (jnp.transpose(fc0_w2), jnp.transpose(fc1_w2),
           jnp.transpose(fc2_w2))

    hbt = None
    for layer in range(3):
        if layer == 0:
            g_list = _emb_gather_call()(xi, emb_table, src)
            outs = _edge_kernel0(eat, g_list, w1t[0], w2t[0])
            msg_list, hbt = outs[:D], outs[D]
        else:
            g_list = _gather_call()(h, src)
            msg_list = _edge_kernel(hbt, g_list, w1t[layer], w2t[layer])
        if layer < 2:
            parts = _scatter_call()(*msg_list, dst, zeros_rows)
            parts_flat = parts.reshape(_NC, N_NODES * D)
            h = _combine(parts_flat[0], parts_flat[1]).reshape(N_NODES, D)
        else:
            pc_list = _scatter_t_call()(*msg_list, dst, zeros_rows)

    out_t = _head(pc_list, jnp.transpose(head_w1), jnp.transpose(head_w2))
    return jnp.transpose(out_t)
